# Initial kernel scaffold; baseline (speedup 1.0000x reference)
import jax
import jax.numpy as jnp
from jax.experimental import pallas as pl


def kernel(nodes, edge_index, etypes, node_feat, bases0, comp0, wself0, bias0, gamma0, beta0, bases1, comp1, wself1, bias1, gamma1, beta1):
    raise NotImplementedError("write your pallas kernel here")


# speedup vs baseline: 2.3204x; 2.3204x over previous
import functools

import jax
import jax.numpy as jnp
from jax import lax
from jax.experimental import pallas as pl
from jax.experimental.pallas import tpu as pltpu
from jax.experimental.pallas import tpu_sc as plsc

N = 10000
E = 320000
R = 16
NB = 4
D = 128

CHUNK = 80
NPAD = 10240
ZROWS = 128


def _combine_body(comp_ref, bases_ref, out_ref):
    out_ref[...] = jnp.dot(comp_ref[...], bases_ref[...],
                           preferred_element_type=jnp.float32)


def _combine(comp, bases):
    return pl.pallas_call(
        _combine_body,
        out_shape=jax.ShapeDtypeStruct((R, D * D), jnp.float32),
    )(comp, bases.reshape(NB, D * D))


_BM = 1000


def _project_body(h_ref, wcat_ref, wself_ref, hw_ref, hself_ref):
    h = h_ref[...]
    hw_ref[...] = jnp.dot(h, wcat_ref[...], preferred_element_type=jnp.float32)
    hself_ref[...] = jnp.dot(h, wself_ref[...],
                             preferred_element_type=jnp.float32)


def _project(h, wcat, wself):
    return pl.pallas_call(
        _project_body,
        grid=(N // _BM,),
        in_specs=[
            pl.BlockSpec((_BM, D), lambda i: (i, 0)),
            pl.BlockSpec((D, R * D), lambda i: (0, 0)),
            pl.BlockSpec((D, D), lambda i: (0, 0)),
        ],
        out_specs=[
            pl.BlockSpec((_BM, R * D), lambda i: (i, 0)),
            pl.BlockSpec((_BM, D), lambda i: (i, 0)),
        ],
        out_shape=[
            jax.ShapeDtypeStruct((N, R * D), jnp.float32),
            jax.ShapeDtypeStruct((N, D), jnp.float32),
        ],
    )(h, wcat, wself)


def _gidx_body(src_ref, et_ref, out_ref):
    out_ref[...] = src_ref[...] * R + et_ref[...]


def _gidx(src, et):
    g = pl.pallas_call(
        _gidx_body,
        out_shape=jax.ShapeDtypeStruct((E // D, D), jnp.int32),
    )(src.reshape(E // D, D), et.reshape(E // D, D))
    return g.reshape(E)


def _finish_body(relu, agg_ref, hself_ref, bias_ref, gamma_ref, beta_ref,
                 out_ref):
    x = jnp.sum(agg_ref[...], axis=0) + hself_ref[...] + bias_ref[...]
    mu = jnp.mean(x, axis=-1, keepdims=True)
    xc = x - mu
    var = jnp.mean(xc * xc, axis=-1, keepdims=True)
    y = gamma_ref[...] * xc / jnp.sqrt(var + 1e-5) + beta_ref[...]
    if relu:
        y = jnp.maximum(y, 0.0)
    out_ref[...] = y


def _finish(agg, hself, bias, gamma, beta, relu):
    nc = agg.shape[0]
    body = functools.partial(_finish_body, relu)
    return pl.pallas_call(
        body,
        grid=(N // _BM,),
        in_specs=[
            pl.BlockSpec((nc, _BM, D), lambda i: (0, i, 0)),
            pl.BlockSpec((_BM, D), lambda i: (i, 0)),
            pl.BlockSpec((1, D), lambda i: (0, 0)),
            pl.BlockSpec((1, D), lambda i: (0, 0)),
            pl.BlockSpec((1, D), lambda i: (0, 0)),
        ],
        out_specs=pl.BlockSpec((_BM, D), lambda i: (i, 0)),
        out_shape=jax.ShapeDtypeStruct((N, D), jnp.float32),
    )(agg, hself, bias.reshape(1, D), gamma.reshape(1, D),
      beta.reshape(1, D))


def _sc_mesh():
    return plsc.VectorSubcoreMesh(core_axis_name="c", subcore_axis_name="s")


def _sc_aggregate(hw_flat, gidx, dst):
    mesh = _sc_mesh()
    nc, ns = mesh.num_cores, mesh.num_subcores
    edges_per_w = E // (nc * ns)
    n_chunks = edges_per_w // CHUNK
    rows_per_sub = NPAD // ns
    assert edges_per_w % CHUNK == 0 and rows_per_sub % ZROWS == 0

    @functools.partial(
        pl.kernel,
        out_type=jax.ShapeDtypeStruct((nc, NPAD, D), jnp.float32),
        mesh=mesh,
        scratch_types=[
            pltpu.VMEM((CHUNK,), jnp.int32),
            pltpu.VMEM((CHUNK,), jnp.int32),
            pltpu.VMEM((CHUNK, D), jnp.float32),
            pltpu.VMEM((ZROWS, D), jnp.float32),
            pltpu.VMEM_SHARED((NPAD, D), jnp.float32),
            pltpu.SemaphoreType.DMA,
        ],
    )
    def k(hw_hbm, gidx_hbm, dst_hbm, out_hbm, idxv, dstv, msgv, zbuf, acc,
          sem):
        c = lax.axis_index("c")
        s = lax.axis_index("s")
        wid = c * ns + s

        @pl.loop(0, ZROWS)
        def _(i):
            @pl.loop(0, D, step=16)
            def _(j):
                zbuf[i, pl.ds(j, 16)] = jnp.zeros((16,), jnp.float32)

        @pl.loop(0, rows_per_sub // ZROWS)
        def _(j):
            pltpu.sync_copy(
                zbuf, acc.at[pl.ds(s * rows_per_sub + j * ZROWS, ZROWS)])

        plsc.subcore_barrier()

        base0 = wid * edges_per_w

        @pl.loop(0, n_chunks)
        def _(j):
            base = base0 + j * CHUNK
            pltpu.sync_copy(gidx_hbm.at[pl.ds(base, CHUNK)], idxv)
            pltpu.sync_copy(dst_hbm.at[pl.ds(base, CHUNK)], dstv)
            pltpu.async_copy(hw_hbm.at[idxv], msgv, sem).wait()
            pltpu.sync_copy(msgv, acc.at[dstv], add=True)

        plsc.subcore_barrier()

        pltpu.sync_copy(
            acc.at[pl.ds(s * rows_per_sub, rows_per_sub)],
            out_hbm.at[c, pl.ds(s * rows_per_sub, rows_per_sub)])

    return k(hw_flat, gidx, dst)


def _sc_gather_rows(table, idx):
    mesh = _sc_mesh()
    nc, ns = mesh.num_cores, mesh.num_subcores
    nw = nc * ns
    n_chunks = N // CHUNK
    per_w = (n_chunks + nw - 1) // nw

    @functools.partial(
        pl.kernel,
        out_type=jax.ShapeDtypeStruct((N, D), jnp.float32),
        mesh=mesh,
        scratch_types=[
            pltpu.VMEM((CHUNK,), jnp.int32),
            pltpu.VMEM((CHUNK, D), jnp.float32),
            pltpu.SemaphoreType.DMA,
        ],
    )
    def k(tab_hbm, idx_hbm, out_hbm, idxv, rowsv, sem):
        c = lax.axis_index("c")
        s = lax.axis_index("s")
        wid = c * ns + s

        @pl.loop(0, per_w)
        def _(j):
            cid = wid + j * nw

            @pl.when(cid < n_chunks)
            def _():
                base = cid * CHUNK
                pltpu.sync_copy(idx_hbm.at[pl.ds(base, CHUNK)], idxv)
                pltpu.async_copy(tab_hbm.at[idxv], rowsv, sem).wait()
                pltpu.sync_copy(rowsv, out_hbm.at[pl.ds(base, CHUNK)])

    return k(table, idx)


def kernel(nodes, edge_index, etypes, node_feat, bases0, comp0, wself0,
           bias0, gamma0, beta0, bases1, comp1, wself1, bias1, gamma1,
           beta1):
    src = edge_index[0].astype(jnp.int32)
    dst = edge_index[1].astype(jnp.int32)
    et = etypes.astype(jnp.int32)
    gidx = _gidx(src, et)

    h = node_feat
    layers = (
        (bases0, comp0, wself0, bias0, gamma0, beta0, True),
        (bases1, comp1, wself1, bias1, gamma1, beta1, False),
    )
    for bases, comp, wself, bias, gamma, beta, relu in layers:
        w2 = _combine(comp, bases)
        wcat = w2.reshape(R, D, D).transpose(1, 0, 2).reshape(D, R * D)
        hw, hself = _project(h, wcat, wself)
        agg = _sc_aggregate(hw.reshape(N * R, D), gidx, dst)
        h = _finish(agg, hself, bias, gamma, beta, relu)

    return _sc_gather_rows(h, nodes.astype(jnp.int32))


# speedup vs baseline: 3.9573x; 1.7055x over previous
import functools

import jax
import jax.numpy as jnp
from jax import lax
from jax.experimental import pallas as pl
from jax.experimental.pallas import tpu as pltpu
from jax.experimental.pallas import tpu_sc as plsc

N = 10000
E = 320000
R = 16
NB = 4
D = 128

CHUNK = 80
NPAD = 10240
ZROWS = 128


def _combine_body(comp_ref, bases_ref, out_ref):
    out_ref[...] = jnp.dot(comp_ref[...], bases_ref[...],
                           preferred_element_type=jnp.float32)


def _combine(comp, bases):
    return pl.pallas_call(
        _combine_body,
        out_shape=jax.ShapeDtypeStruct((R, D * D), jnp.float32),
    )(comp, bases.reshape(NB, D * D))


_BM = 1000


def _project_body(h_ref, wcat_ref, wself_ref, hw_ref, hself_ref):
    h = h_ref[...]
    hw_ref[...] = jnp.dot(h, wcat_ref[...], preferred_element_type=jnp.float32)
    hself_ref[...] = jnp.dot(h, wself_ref[...],
                             preferred_element_type=jnp.float32)


def _project(h, wcat, wself):
    return pl.pallas_call(
        _project_body,
        grid=(N // _BM,),
        in_specs=[
            pl.BlockSpec((_BM, D), lambda i: (i, 0)),
            pl.BlockSpec((D, R * D), lambda i: (0, 0)),
            pl.BlockSpec((D, D), lambda i: (0, 0)),
        ],
        out_specs=[
            pl.BlockSpec((_BM, R * D), lambda i: (i, 0)),
            pl.BlockSpec((_BM, D), lambda i: (i, 0)),
        ],
        out_shape=[
            jax.ShapeDtypeStruct((N, R * D), jnp.float32),
            jax.ShapeDtypeStruct((N, D), jnp.float32),
        ],
    )(h, wcat, wself)


def _gidx_body(src_ref, et_ref, out_ref):
    out_ref[...] = src_ref[...] * R + et_ref[...]


def _gidx(src, et):
    g = pl.pallas_call(
        _gidx_body,
        out_shape=jax.ShapeDtypeStruct((E // D, D), jnp.int32),
    )(src.reshape(E // D, D), et.reshape(E // D, D))
    return g.reshape(E)


def _finish_body(relu, agg_ref, hself_ref, bias_ref, gamma_ref, beta_ref,
                 out_ref):
    x = jnp.sum(agg_ref[...], axis=0) + hself_ref[...] + bias_ref[...]
    mu = jnp.mean(x, axis=-1, keepdims=True)
    xc = x - mu
    var = jnp.mean(xc * xc, axis=-1, keepdims=True)
    y = gamma_ref[...] * xc / jnp.sqrt(var + 1e-5) + beta_ref[...]
    if relu:
        y = jnp.maximum(y, 0.0)
    out_ref[...] = y


def _finish(agg, hself, bias, gamma, beta, relu):
    nc = agg.shape[0]
    body = functools.partial(_finish_body, relu)
    return pl.pallas_call(
        body,
        grid=(N // _BM,),
        in_specs=[
            pl.BlockSpec((nc, _BM, D), lambda i: (0, i, 0)),
            pl.BlockSpec((_BM, D), lambda i: (i, 0)),
            pl.BlockSpec((1, D), lambda i: (0, 0)),
            pl.BlockSpec((1, D), lambda i: (0, 0)),
            pl.BlockSpec((1, D), lambda i: (0, 0)),
        ],
        out_specs=pl.BlockSpec((_BM, D), lambda i: (i, 0)),
        out_shape=jax.ShapeDtypeStruct((N, D), jnp.float32),
    )(agg, hself, bias.reshape(1, D), gamma.reshape(1, D),
      beta.reshape(1, D))


def _sc_mesh():
    return plsc.VectorSubcoreMesh(core_axis_name="c", subcore_axis_name="s")


def _sc_aggregate(hw_flat, gidx, dst):
    mesh = _sc_mesh()
    nc, ns = mesh.num_cores, mesh.num_subcores
    edges_per_w = E // (nc * ns)
    n_chunks = edges_per_w // CHUNK
    rows_per_sub = NPAD // ns
    assert edges_per_w % CHUNK == 0 and rows_per_sub % ZROWS == 0

    @functools.partial(
        pl.kernel,
        out_type=jax.ShapeDtypeStruct((nc, NPAD, D), jnp.float32),
        mesh=mesh,
        scratch_types=[
            pltpu.VMEM((edges_per_w,), jnp.int32),
            pltpu.VMEM((edges_per_w,), jnp.int32),
            pltpu.VMEM((CHUNK, D), jnp.float32),
            pltpu.VMEM((CHUNK, D), jnp.float32),
            pltpu.VMEM_SHARED((NPAD, D), jnp.float32),
            pltpu.SemaphoreType.DMA,
            pltpu.SemaphoreType.DMA,
        ],
    )
    def k(hw_hbm, gidx_hbm, dst_hbm, out_hbm, idxv, dstv, msg0, msg1,
          acc, sem0, sem1):
        c = lax.axis_index("c")
        s = lax.axis_index("s")
        wid = c * ns + s

        @pl.loop(0, CHUNK)
        def _(i):
            @pl.loop(0, D, step=16)
            def _(j):
                msg0[i, pl.ds(j, 16)] = jnp.zeros((16,), jnp.float32)

        base0 = wid * edges_per_w
        pltpu.sync_copy(gidx_hbm.at[pl.ds(base0, edges_per_w)], idxv)
        pltpu.sync_copy(dst_hbm.at[pl.ds(base0, edges_per_w)], dstv)

        @pl.loop(0, rows_per_sub // CHUNK)
        def _(j):
            pltpu.sync_copy(
                msg0, acc.at[pl.ds(s * rows_per_sub + j * CHUNK, CHUNK)])

        plsc.subcore_barrier()

        def fire(chunk, buf, sem):
            idx_sl = idxv.at[pl.ds(chunk * CHUNK, CHUNK)]
            return pltpu.async_copy(hw_hbm.at[idx_sl], buf, sem)

        def scatter(chunk, buf):
            dst_sl = dstv.at[pl.ds(chunk * CHUNK, CHUNK)]
            pltpu.sync_copy(buf, acc.at[dst_sl], add=True)

        d0 = fire(0, msg0, sem0)

        @pl.loop(0, (n_chunks - 1) // 2)
        def _(t):
            c0 = 2 * t
            d1 = fire(c0 + 1, msg1, sem1)
            d0.wait()
            scatter(c0, msg0)
            d0b = fire(c0 + 2, msg0, sem0)
            d1.wait()
            scatter(c0 + 1, msg1)

        d0.wait()
        scatter(n_chunks - 1, msg0)

        plsc.subcore_barrier()

        pltpu.sync_copy(
            acc.at[pl.ds(s * rows_per_sub, rows_per_sub)],
            out_hbm.at[c, pl.ds(s * rows_per_sub, rows_per_sub)])

    return k(hw_flat, gidx, dst)


def _sc_gather_rows(table, idx):
    mesh = _sc_mesh()
    nc, ns = mesh.num_cores, mesh.num_subcores
    nw = nc * ns
    n_chunks = N // CHUNK
    per_w = (n_chunks + nw - 1) // nw

    @functools.partial(
        pl.kernel,
        out_type=jax.ShapeDtypeStruct((N, D), jnp.float32),
        mesh=mesh,
        scratch_types=[
            pltpu.VMEM((CHUNK,), jnp.int32),
            pltpu.VMEM((CHUNK, D), jnp.float32),
            pltpu.SemaphoreType.DMA,
        ],
    )
    def k(tab_hbm, idx_hbm, out_hbm, idxv, rowsv, sem):
        c = lax.axis_index("c")
        s = lax.axis_index("s")
        wid = c * ns + s

        @pl.loop(0, per_w)
        def _(j):
            cid = wid + j * nw

            @pl.when(cid < n_chunks)
            def _():
                base = cid * CHUNK
                pltpu.sync_copy(idx_hbm.at[pl.ds(base, CHUNK)], idxv)
                pltpu.async_copy(tab_hbm.at[idxv], rowsv, sem).wait()
                pltpu.sync_copy(rowsv, out_hbm.at[pl.ds(base, CHUNK)])

    return k(table, idx)


def kernel(nodes, edge_index, etypes, node_feat, bases0, comp0, wself0,
           bias0, gamma0, beta0, bases1, comp1, wself1, bias1, gamma1,
           beta1):
    src = edge_index[0].astype(jnp.int32)
    dst = edge_index[1].astype(jnp.int32)
    et = etypes.astype(jnp.int32)
    gidx = _gidx(src, et)

    h = node_feat
    layers = (
        (bases0, comp0, wself0, bias0, gamma0, beta0, True),
        (bases1, comp1, wself1, bias1, gamma1, beta1, False),
    )
    for bases, comp, wself, bias, gamma, beta, relu in layers:
        w2 = _combine(comp, bases)
        wcat = w2.reshape(R, D, D).transpose(1, 0, 2).reshape(D, R * D)
        hw, hself = _project(h, wcat, wself)
        agg = _sc_aggregate(hw.reshape(N * R, D), gidx, dst)
        h = _finish(agg, hself, bias, gamma, beta, relu)

    return _sc_gather_rows(h, nodes.astype(jnp.int32))


# speedup vs baseline: 4.0342x; 1.0194x over previous
import functools

import jax
import jax.numpy as jnp
from jax import lax
from jax.experimental import pallas as pl
from jax.experimental.pallas import tpu as pltpu
from jax.experimental.pallas import tpu_sc as plsc

N = 10000
E = 320000
R = 16
NB = 4
D = 128

CHUNK = 96
TAIL = 16
NPAD = 10240


def _combine_body(comp_ref, bases_ref, out_ref):
    out_ref[...] = jnp.dot(comp_ref[...], bases_ref[...],
                           preferred_element_type=jnp.float32)


def _combine(comp, bases):
    return pl.pallas_call(
        _combine_body,
        out_shape=jax.ShapeDtypeStruct((R, D * D), jnp.float32),
    )(comp, bases.reshape(NB, D * D))


_BM = 1000


def _project_body(h_ref, wcat_ref, wself_ref, hw_ref, hself_ref):
    h = h_ref[...]
    hw_ref[...] = jnp.dot(h, wcat_ref[...], preferred_element_type=jnp.float32)
    hself_ref[...] = jnp.dot(h, wself_ref[...],
                             preferred_element_type=jnp.float32)


def _project(h, wcat, wself):
    return pl.pallas_call(
        _project_body,
        grid=(N // _BM,),
        in_specs=[
            pl.BlockSpec((_BM, D), lambda i: (i, 0)),
            pl.BlockSpec((D, R * D), lambda i: (0, 0)),
            pl.BlockSpec((D, D), lambda i: (0, 0)),
        ],
        out_specs=[
            pl.BlockSpec((_BM, R * D), lambda i: (i, 0)),
            pl.BlockSpec((_BM, D), lambda i: (i, 0)),
        ],
        out_shape=[
            jax.ShapeDtypeStruct((N, R * D), jnp.float32),
            jax.ShapeDtypeStruct((N, D), jnp.float32),
        ],
        compiler_params=pltpu.CompilerParams(
            dimension_semantics=("parallel",)),
    )(h, wcat, wself)


def _gidx_body(src_ref, et_ref, out_ref):
    out_ref[...] = src_ref[...] * R + et_ref[...]


def _gidx(src, et):
    g = pl.pallas_call(
        _gidx_body,
        out_shape=jax.ShapeDtypeStruct((E // D, D), jnp.int32),
    )(src.reshape(E // D, D), et.reshape(E // D, D))
    return g.reshape(E)


def _finish_body(relu, agg_ref, hself_ref, bias_ref, gamma_ref, beta_ref,
                 out_ref):
    x = jnp.sum(agg_ref[...], axis=0) + hself_ref[...] + bias_ref[...]
    mu = jnp.mean(x, axis=-1, keepdims=True)
    xc = x - mu
    var = jnp.mean(xc * xc, axis=-1, keepdims=True)
    y = gamma_ref[...] * xc / jnp.sqrt(var + 1e-5) + beta_ref[...]
    if relu:
        y = jnp.maximum(y, 0.0)
    out_ref[...] = y


def _finish(agg, hself, bias, gamma, beta, relu):
    nc = agg.shape[0]
    body = functools.partial(_finish_body, relu)
    return pl.pallas_call(
        body,
        grid=(N // _BM,),
        in_specs=[
            pl.BlockSpec((nc, _BM, D), lambda i: (0, i, 0)),
            pl.BlockSpec((_BM, D), lambda i: (i, 0)),
            pl.BlockSpec((1, D), lambda i: (0, 0)),
            pl.BlockSpec((1, D), lambda i: (0, 0)),
            pl.BlockSpec((1, D), lambda i: (0, 0)),
        ],
        out_specs=pl.BlockSpec((_BM, D), lambda i: (i, 0)),
        out_shape=jax.ShapeDtypeStruct((N, D), jnp.float32),
        compiler_params=pltpu.CompilerParams(
            dimension_semantics=("parallel",)),
    )(agg, hself, bias.reshape(1, D), gamma.reshape(1, D),
      beta.reshape(1, D))


def _sc_mesh():
    return plsc.VectorSubcoreMesh(core_axis_name="c", subcore_axis_name="s")


def _sc_aggregate(hw_flat, gidx, dst):
    mesh = _sc_mesh()
    nc, ns = mesh.num_cores, mesh.num_subcores
    edges_per_w = E // (nc * ns)
    n_full = edges_per_w // CHUNK
    rows_per_sub = NPAD // ns
    assert edges_per_w == n_full * CHUNK + TAIL
    assert rows_per_sub % 80 == 0 and n_full % 2 == 0

    @functools.partial(
        pl.kernel,
        out_type=jax.ShapeDtypeStruct((nc, NPAD, D), jnp.float32),
        mesh=mesh,
        scratch_types=[
            pltpu.VMEM((edges_per_w,), jnp.int32),
            pltpu.VMEM((edges_per_w,), jnp.int32),
            pltpu.VMEM((CHUNK, D), jnp.float32),
            pltpu.VMEM((CHUNK, D), jnp.float32),
            pltpu.VMEM_SHARED((NPAD, D), jnp.float32),
            pltpu.SemaphoreType.DMA,
            pltpu.SemaphoreType.DMA,
        ],
    )
    def k(hw_hbm, gidx_hbm, dst_hbm, out_hbm, idxv, dstv, msg0, msg1,
          acc, sem0, sem1):
        c = lax.axis_index("c")
        s = lax.axis_index("s")
        wid = c * ns + s

        @pl.loop(0, CHUNK)
        def _(i):
            @pl.loop(0, D, step=16)
            def _(j):
                msg0[i, pl.ds(j, 16)] = jnp.zeros((16,), jnp.float32)

        base0 = wid * edges_per_w
        pltpu.sync_copy(gidx_hbm.at[pl.ds(base0, edges_per_w)], idxv)
        pltpu.sync_copy(dst_hbm.at[pl.ds(base0, edges_per_w)], dstv)

        @pl.loop(0, rows_per_sub // 80)
        def _(j):
            pltpu.sync_copy(
                msg0.at[pl.ds(0, 80)],
                acc.at[pl.ds(s * rows_per_sub + j * 80, 80)])

        plsc.subcore_barrier()

        def fire(chunk, buf, sem):
            idx_sl = idxv.at[pl.ds(chunk * CHUNK, CHUNK)]
            return pltpu.async_copy(hw_hbm.at[idx_sl], buf, sem)

        def scatter(chunk, buf):
            dst_sl = dstv.at[pl.ds(chunk * CHUNK, CHUNK)]
            pltpu.sync_copy(buf, acc.at[dst_sl], add=True)

        d0 = fire(0, msg0, sem0)

        @pl.loop(0, n_full // 2)
        def _(t):
            c0 = 2 * t
            d1 = fire(c0 + 1, msg1, sem1)
            d0.wait()
            scatter(c0, msg0)

            @pl.when(c0 + 2 < n_full)
            def _():
                fire(c0 + 2, msg0, sem0)

            d1.wait()
            scatter(c0 + 1, msg1)

        tb = n_full * CHUNK
        pltpu.async_copy(
            hw_hbm.at[idxv.at[pl.ds(tb, TAIL)]],
            msg0.at[pl.ds(0, TAIL)], sem0).wait()
        pltpu.sync_copy(msg0.at[pl.ds(0, TAIL)],
                        acc.at[dstv.at[pl.ds(tb, TAIL)]], add=True)

        plsc.subcore_barrier()

        pltpu.sync_copy(
            acc.at[pl.ds(s * rows_per_sub, rows_per_sub)],
            out_hbm.at[c, pl.ds(s * rows_per_sub, rows_per_sub)])

    return k(hw_flat, gidx, dst)


def _sc_gather_rows(table, idx):
    mesh = _sc_mesh()
    nc, ns = mesh.num_cores, mesh.num_subcores
    nw = nc * ns
    gch = 80
    n_chunks = N // gch
    assert n_chunks * gch == N
    per_w = (n_chunks + nw - 1) // nw

    @functools.partial(
        pl.kernel,
        out_type=jax.ShapeDtypeStruct((N, D), jnp.float32),
        mesh=mesh,
        scratch_types=[
            pltpu.VMEM((gch,), jnp.int32),
            pltpu.VMEM((gch, D), jnp.float32),
            pltpu.SemaphoreType.DMA,
        ],
    )
    def k(tab_hbm, idx_hbm, out_hbm, idxv, rowsv, sem):
        c = lax.axis_index("c")
        s = lax.axis_index("s")
        wid = c * ns + s

        @pl.loop(0, per_w)
        def _(j):
            cid = wid + j * nw

            @pl.when(cid < n_chunks)
            def _():
                base = cid * gch
                pltpu.sync_copy(idx_hbm.at[pl.ds(base, gch)], idxv)
                pltpu.async_copy(tab_hbm.at[idxv], rowsv, sem).wait()
                pltpu.sync_copy(rowsv, out_hbm.at[pl.ds(base, gch)])

    return k(table, idx)


def kernel(nodes, edge_index, etypes, node_feat, bases0, comp0, wself0,
           bias0, gamma0, beta0, bases1, comp1, wself1, bias1, gamma1,
           beta1):
    src = edge_index[0].astype(jnp.int32)
    dst = edge_index[1].astype(jnp.int32)
    et = etypes.astype(jnp.int32)
    gidx = _gidx(src, et)

    h = node_feat
    layers = (
        (bases0, comp0, wself0, bias0, gamma0, beta0, True),
        (bases1, comp1, wself1, bias1, gamma1, beta1, False),
    )
    for bases, comp, wself, bias, gamma, beta, relu in layers:
        w2 = _combine(comp, bases)
        wcat = w2.reshape(R, D, D).transpose(1, 0, 2).reshape(D, R * D)
        hw, hself = _project(h, wcat, wself)
        agg = _sc_aggregate(hw.reshape(N * R, D), gidx, dst)
        h = _finish(agg, hself, bias, gamma, beta, relu)

    return _sc_gather_rows(h, nodes.astype(jnp.int32))
